# baseline (speedup 1.0000x reference)
import functools

import jax
import jax.numpy as jnp
from jax import lax
from jax.experimental import pallas as pl
from jax.experimental.pallas import tpu as pltpu
from jax.experimental.pallas import tpu_sc as plsc

N = 50000
NP = 50176
E = 800000
F = 32
NC = 2
NS = 16
NW = NC * NS
EPW = 25088
EP = EPW * NW
CHUNK = 128
NCHUNK = EPW // CHUNK
IB = 28
STRIPE = NP // NS
ZROWS = 196


@functools.cache
def _get_mesh():
    return plsc.VectorSubcoreMesh(core_axis_name="c", subcore_axis_name="s",
                                  num_cores=NC, num_subcores=NS)


def _deg_body(dst_hbm, deg_out, idx_v, hist_v):
    cid = lax.axis_index("c")
    sid = lax.axis_index("s")
    wid = cid * NS + sid
    z16 = jnp.zeros((16,), jnp.float32)
    o16 = jnp.ones((16,), jnp.float32)

    def zloop(i, _):
        for m in range(8):
            hist_v[pl.ds(i * 128 + m * 16, 16)] = z16
        return 0
    lax.fori_loop(0, NP // 128, zloop, 0)

    pltpu.sync_copy(dst_hbm.at[pl.ds(wid * EPW, EPW)], idx_v)

    def hloop(i, _):
        for m in range(4):
            idx = idx_v[pl.ds(i * 64 + m * 16, 16)]
            plsc.addupdate_scatter(hist_v, [idx], o16)
        return 0
    lax.fori_loop(0, EPW // 64, hloop, 0)

    pltpu.sync_copy(hist_v, deg_out.at[wid])


@functools.partial(jax.jit)
def _deg_kernel(dst_flat):
    return pl.kernel(
        _deg_body,
        out_type=jax.ShapeDtypeStruct((NW, NP), jnp.float32),
        mesh=_get_mesh(),
        compiler_params=pltpu.CompilerParams(needs_layout_passes=False,
                                             use_tc_tiling_on_sc=False),
        scratch_types=[
            pltpu.VMEM((EPW,), jnp.int32),
            pltpu.VMEM((NP,), jnp.float32),
        ],
    )(dst_flat)


def _scatter_body(src2d, dst2d, y_hbm, s_out, s_sh, idx_s, idx_d, rows, zbuf,
                  gs0, gs1, gs2, gs3, ss0, ss1, ss2, ss3):
    gsem = (gs0, gs1, gs2, gs3)
    ssem = (ss0, ss1, ss2, ss3)
    cid = lax.axis_index("c")
    sid = lax.axis_index("s")
    wid = cid * NS + sid
    z16 = jnp.zeros((16,), jnp.float32)

    def zloop(i, _):
        zbuf[i, pl.ds(0, 16)] = z16
        zbuf[i, pl.ds(16, 16)] = z16
        return 0
    lax.fori_loop(0, ZROWS, zloop, 0)
    for k in range(STRIPE // ZROWS):
        pltpu.async_copy(zbuf, s_sh.at[pl.ds(sid * STRIPE + k * ZROWS, ZROWS)],
                         ssem[k % 4])
    for k in range(STRIPE // ZROWS):
        pltpu.make_async_copy(
            zbuf, s_sh.at[pl.ds(sid * STRIPE + k * ZROWS, ZROWS)],
            ssem[k % 4]).wait()
    plsc.subcore_barrier()

    def block(blk, _):
        pltpu.sync_copy(src2d.at[wid, pl.ds(blk * IB, IB)], idx_s)
        pltpu.sync_copy(dst2d.at[wid, pl.ds(blk * IB, IB)], idx_d)
        for b in range(3):
            pltpu.async_copy(y_hbm.at[idx_s.at[b]], rows.at[b], gsem[b])

        def grp(k, _):
            for bb in range(4):
                j = 4 * k + bb
                pltpu.make_async_copy(y_hbm.at[idx_s.at[j]], rows.at[bb],
                                      gsem[bb]).wait()
                pltpu.async_copy(rows.at[bb], s_sh.at[idx_d.at[j]], ssem[bb],
                                 add=True)
                bn = (bb + 3) % 4

                @pl.when(j + 3 < IB)
                def _():
                    @pl.when(j >= 1)
                    def _():
                        pltpu.make_async_copy(rows.at[bn],
                                              s_sh.at[idx_d.at[j - 1]],
                                              ssem[bn]).wait()
                    pltpu.async_copy(y_hbm.at[idx_s.at[j + 3]], rows.at[bn],
                                     gsem[bn])
            return 0
        lax.fori_loop(0, IB // 4, grp, 0)
        for b in range(4):
            pltpu.make_async_copy(rows.at[b], s_sh.at[idx_d.at[IB - 4 + b]],
                                  ssem[b]).wait()
        return 0
    lax.fori_loop(0, NCHUNK // IB, block, 0)

    plsc.subcore_barrier()
    pltpu.sync_copy(s_sh.at[pl.ds(sid * STRIPE, STRIPE)],
                    s_out.at[cid, pl.ds(sid * STRIPE, STRIPE)])


@functools.partial(jax.jit)
def _scatter_kernel(src2d, dst2d, y):
    return pl.kernel(
        _scatter_body,
        out_type=jax.ShapeDtypeStruct((NC, NP, F), jnp.float32),
        mesh=_get_mesh(),
        compiler_params=pltpu.CompilerParams(needs_layout_passes=False,
                                             use_tc_tiling_on_sc=False),
        scratch_types=[
            pltpu.VMEM_SHARED((NP, F), jnp.float32),
            pltpu.VMEM((IB, CHUNK), jnp.int32),
            pltpu.VMEM((IB, CHUNK), jnp.int32),
            pltpu.VMEM((4, CHUNK, F), jnp.float32),
            pltpu.VMEM((ZROWS, F), jnp.float32),
            pltpu.SemaphoreType.DMA,
            pltpu.SemaphoreType.DMA,
            pltpu.SemaphoreType.DMA,
            pltpu.SemaphoreType.DMA,
            pltpu.SemaphoreType.DMA,
            pltpu.SemaphoreType.DMA,
            pltpu.SemaphoreType.DMA,
            pltpu.SemaphoreType.DMA,
        ],
    )(src2d, dst2d, y)


BLK_P = 1024
NM = NP // 4
BLK_E = 1568
BLK_M = 2000


def _prep_body(x_ref, pe_ref, deg_ref, et_ref, tw_ref, tb_ref, w1_ref,
               y1_ref, dx_ref):
    xb = x_ref[...].astype(jnp.int32)
    rows = lax.broadcasted_iota(jnp.int32, (28, BLK_P), 0)
    oh = (rows == xb).astype(jnp.float32)
    h0 = jnp.dot(et_ref[...], oh, preferred_element_type=jnp.float32)
    h0 += jnp.dot(tw_ref[...], pe_ref[...], preferred_element_type=jnp.float32)
    h0 += tb_ref[...]
    tT = jnp.dot(w1_ref[...], h0, preferred_element_type=jnp.float32)
    deg = jnp.sum(deg_ref[...], axis=0, keepdims=True) + 1.0
    dinv = lax.rsqrt(deg)
    y1T = tT * dinv
    dxT = jnp.broadcast_to(dinv, (F, BLK_P))

    def merge(vT):
        v3 = jnp.transpose(vT).reshape(BLK_P // 4, 4, F)
        return jnp.concatenate([v3[:, m, :] for m in range(4)], axis=1)

    y1_ref[...] = merge(y1T)
    dx_ref[...] = merge(dxT)


@functools.partial(jax.jit)
def _prep_kernel(xf, peT, deg_parts, EtT, TwT, tb_col, W1T):
    grid = NP // BLK_P
    return pl.pallas_call(
        _prep_body,
        grid=(grid,),
        in_specs=[
            pl.BlockSpec((1, BLK_P), lambda i: (0, i)),
            pl.BlockSpec((5, BLK_P), lambda i: (0, i)),
            pl.BlockSpec((NW, BLK_P), lambda i: (0, i)),
            pl.BlockSpec((64, 28), lambda i: (0, 0)),
            pl.BlockSpec((64, 5), lambda i: (0, 0)),
            pl.BlockSpec((64, 1), lambda i: (0, 0)),
            pl.BlockSpec((F, 64), lambda i: (0, 0)),
        ],
        out_specs=[
            pl.BlockSpec((BLK_P // 4, 128), lambda i: (i, 0)),
            pl.BlockSpec((BLK_P // 4, 128), lambda i: (i, 0)),
        ],
        out_shape=[
            jax.ShapeDtypeStruct((NM, 128), jnp.float32),
            jax.ShapeDtypeStruct((NM, 128), jnp.float32),
        ],
    )(xf, peT, deg_parts, EtT, TwT, tb_col, W1T)


def _ef_body(sp_ref, y1_ref, dx_ref, b1_ref, y2_ref):
    s = sp_ref[0] + sp_ref[1]
    dx = dx_ref[...]
    h1 = jnp.maximum(dx * (s + y1_ref[...]) + b1_ref[...], 0.0)
    y2_ref[...] = dx * h1


@functools.partial(jax.jit)
def _ef_kernel(s1m, y1m, dxm, b1_4):
    grid = NM // BLK_E
    return pl.pallas_call(
        _ef_body,
        grid=(grid,),
        in_specs=[
            pl.BlockSpec((NC, BLK_E, 128), lambda i: (0, i, 0)),
            pl.BlockSpec((BLK_E, 128), lambda i: (i, 0)),
            pl.BlockSpec((BLK_E, 128), lambda i: (i, 0)),
            pl.BlockSpec((1, 128), lambda i: (0, 0)),
        ],
        out_specs=pl.BlockSpec((BLK_E, 128), lambda i: (i, 0)),
        out_shape=jax.ShapeDtypeStruct((NM, 128), jnp.float32),
    )(s1m, y1m, dxm, b1_4)


BLK_U = 1792


def _mu_ls_body(sp_ref, y2_ref, dx_ref, wmu_ref, bmu_ref, wls_ref, bls_ref,
                mu_ref, ls_ref):
    agg = dx_ref[...] * (sp_ref[0] + sp_ref[1] + y2_ref[...])

    def unmerge(m4):
        parts = [m4[:, F * m:F * (m + 1)] for m in range(4)]
        return jnp.stack(parts, axis=1).reshape(BLK_U, F)

    mu4 = jnp.dot(agg, wmu_ref[...],
                  preferred_element_type=jnp.float32) + bmu_ref[...]
    ls4 = jnp.dot(agg, wls_ref[...],
                  preferred_element_type=jnp.float32) + bls_ref[...]
    mu_ref[...] = unmerge(mu4)
    ls_ref[...] = unmerge(ls4)


@functools.partial(jax.jit)
def _mu_ls_kernel(s2m, y2m, dxm, W4mu, b4mu, W4ls, b4ls):
    grid = NP // BLK_U
    return pl.pallas_call(
        _mu_ls_body,
        grid=(grid,),
        in_specs=[
            pl.BlockSpec((NC, BLK_U // 4, 128), lambda i: (0, i, 0)),
            pl.BlockSpec((BLK_U // 4, 128), lambda i: (i, 0)),
            pl.BlockSpec((BLK_U // 4, 128), lambda i: (i, 0)),
            pl.BlockSpec((128, 128), lambda i: (0, 0)),
            pl.BlockSpec((1, 128), lambda i: (0, 0)),
            pl.BlockSpec((128, 128), lambda i: (0, 0)),
            pl.BlockSpec((1, 128), lambda i: (0, 0)),
        ],
        out_specs=[
            pl.BlockSpec((BLK_U, F), lambda i: (i, 0)),
            pl.BlockSpec((BLK_U, F), lambda i: (i, 0)),
        ],
        out_shape=[
            jax.ShapeDtypeStruct((NP, F), jnp.float32),
            jax.ShapeDtypeStruct((NP, F), jnp.float32),
        ],
    )(s2m, y2m, dxm, W4mu, b4mu, W4ls, b4ls)


def kernel(x, edge_index, laplacian_eigenvector_pe, embed_table, trans_W,
           trans_b, W1, b1, W_mu, b_mu, W_ls, b_ls):
    xf = jnp.pad(x.reshape(-1).astype(jnp.float32), (0, NP - N))
    xf = xf.reshape(1, NP)
    peT = jnp.pad(laplacian_eigenvector_pe.astype(jnp.float32),
                  ((0, NP - N), (0, 0))).T
    ei = edge_index.astype(jnp.int32)
    pad_row = N + jnp.arange(EP - E, dtype=jnp.int32) % (NP - N)
    pad_e = jnp.broadcast_to(pad_row, (2, EP - E))
    e2 = jnp.concatenate([ei, pad_e], axis=1)
    src2d = e2[0].reshape(NW, NCHUNK, CHUNK)
    dst2d = e2[1].reshape(NW, NCHUNK, CHUNK)
    dst_flat = e2[1]

    EtT = embed_table.astype(jnp.float32).T
    TwT = trans_W.astype(jnp.float32).T
    tb_col = trans_b.astype(jnp.float32).reshape(64, 1)
    W1T = W1.astype(jnp.float32).T
    b1_4 = jnp.tile(b1.astype(jnp.float32), 4).reshape(1, 128)
    eye4 = jnp.eye(4, dtype=jnp.float32)
    W4mu = jnp.kron(eye4, W_mu.astype(jnp.float32))
    W4ls = jnp.kron(eye4, W_ls.astype(jnp.float32))
    b4mu = jnp.tile(b_mu.astype(jnp.float32), 4).reshape(1, 128)
    b4ls = jnp.tile(b_ls.astype(jnp.float32), 4).reshape(1, 128)

    deg_parts = _deg_kernel(dst_flat)
    y1m, dxm = _prep_kernel(xf, peT, deg_parts, EtT, TwT, tb_col, W1T)
    s1p = _scatter_kernel(src2d, dst2d, y1m.reshape(NP, F))
    y2m = _ef_kernel(s1p.reshape(NC, NM, 128), y1m, dxm, b1_4)
    s2p = _scatter_kernel(src2d, dst2d, y2m.reshape(NP, F))
    mu, ls = _mu_ls_kernel(s2p.reshape(NC, NM, 128), y2m, dxm,
                           W4mu, b4mu, W4ls, b4ls)
    return (mu[:N], ls[:N])


# speedup vs baseline: 1.1812x; 1.1812x over previous
import functools

import jax
import jax.numpy as jnp
from jax import lax
from jax.experimental import pallas as pl
from jax.experimental.pallas import tpu as pltpu
from jax.experimental.pallas import tpu_sc as plsc

N = 50000
NP = 50176
E = 800000
F = 32
NC = 2
NS = 16
NW = NC * NS
EPW = 25088
EP = EPW * NW
CHUNK = 128
NCHUNK = EPW // CHUNK
IB = 28
STRIPE = NP // NS
ZROWS = 196


@functools.cache
def _get_mesh():
    return plsc.VectorSubcoreMesh(core_axis_name="c", subcore_axis_name="s",
                                  num_cores=NC, num_subcores=NS)


def _deg_body(dst_hbm, deg_out, idx_v, hist_v):
    cid = lax.axis_index("c")
    sid = lax.axis_index("s")
    wid = cid * NS + sid
    z16 = jnp.zeros((16,), jnp.float32)
    o16 = jnp.ones((16,), jnp.float32)

    def zloop(i, _):
        for m in range(8):
            hist_v[pl.ds(i * 128 + m * 16, 16)] = z16
        return 0
    lax.fori_loop(0, NP // 128, zloop, 0)

    pltpu.sync_copy(dst_hbm.at[pl.ds(wid * EPW, EPW)], idx_v)

    def hloop(i, _):
        for m in range(4):
            idx = idx_v[pl.ds(i * 64 + m * 16, 16)]
            plsc.addupdate_scatter(hist_v, [idx], o16)
        return 0
    lax.fori_loop(0, EPW // 64, hloop, 0)

    pltpu.sync_copy(hist_v, deg_out.at[wid])


@functools.partial(jax.jit)
def _deg_kernel(dst_flat):
    return pl.kernel(
        _deg_body,
        out_type=jax.ShapeDtypeStruct((NW, NP), jnp.float32),
        mesh=_get_mesh(),
        compiler_params=pltpu.CompilerParams(needs_layout_passes=False,
                                             use_tc_tiling_on_sc=False),
        scratch_types=[
            pltpu.VMEM((EPW,), jnp.int32),
            pltpu.VMEM((NP,), jnp.float32),
        ],
    )(dst_flat)


def _scatter_body(src2d, dst2d, y_hbm, s_out, s_sh, idx_s, idx_d, rows, zbuf,
                  gs0, gs1, gs2, gs3, ss0, ss1, ss2, ss3):
    gsem = (gs0, gs1, gs2, gs3)
    ssem = (ss0, ss1, ss2, ss3)
    cid = lax.axis_index("c")
    sid = lax.axis_index("s")
    wid = cid * NS + sid
    z16 = jnp.zeros((16,), jnp.float32)

    def zloop(i, _):
        zbuf[i, pl.ds(0, 16)] = z16
        zbuf[i, pl.ds(16, 16)] = z16
        return 0
    lax.fori_loop(0, ZROWS, zloop, 0)
    for k in range(STRIPE // ZROWS):
        pltpu.async_copy(zbuf, s_sh.at[pl.ds(sid * STRIPE + k * ZROWS, ZROWS)],
                         ssem[k % 4])
    for k in range(STRIPE // ZROWS):
        pltpu.make_async_copy(
            zbuf, s_sh.at[pl.ds(sid * STRIPE + k * ZROWS, ZROWS)],
            ssem[k % 4]).wait()
    plsc.subcore_barrier()

    def block(blk, _):
        pltpu.sync_copy(src2d.at[wid, pl.ds(blk * IB, IB)], idx_s)
        pltpu.sync_copy(dst2d.at[wid, pl.ds(blk * IB, IB)], idx_d)
        for b in range(3):
            pltpu.async_copy(y_hbm.at[idx_s.at[b]], rows.at[b], gsem[b])

        def grp(k, _):
            for bb in range(4):
                j = 4 * k + bb
                pltpu.make_async_copy(y_hbm.at[idx_s.at[j]], rows.at[bb],
                                      gsem[bb]).wait()
                pltpu.async_copy(rows.at[bb], s_sh.at[idx_d.at[j]], ssem[bb],
                                 add=True)
                bn = (bb + 3) % 4

                @pl.when(j + 3 < IB)
                def _():
                    @pl.when(j >= 1)
                    def _():
                        pltpu.make_async_copy(rows.at[bn],
                                              s_sh.at[idx_d.at[j - 1]],
                                              ssem[bn]).wait()
                    pltpu.async_copy(y_hbm.at[idx_s.at[j + 3]], rows.at[bn],
                                     gsem[bn])
            return 0
        lax.fori_loop(0, IB // 4, grp, 0)
        for b in range(4):
            pltpu.make_async_copy(rows.at[b], s_sh.at[idx_d.at[IB - 4 + b]],
                                  ssem[b]).wait()
        return 0
    lax.fori_loop(0, NCHUNK // IB, block, 0)

    plsc.subcore_barrier()
    pltpu.sync_copy(s_sh.at[pl.ds(sid * STRIPE, STRIPE)],
                    s_out.at[cid, pl.ds(sid * STRIPE, STRIPE)])


@functools.partial(jax.jit)
def _scatter_kernel(src2d, dst2d, y):
    return pl.kernel(
        _scatter_body,
        out_type=jax.ShapeDtypeStruct((NC, NP, F), jnp.float32),
        mesh=_get_mesh(),
        compiler_params=pltpu.CompilerParams(needs_layout_passes=False,
                                             use_tc_tiling_on_sc=False),
        scratch_types=[
            pltpu.VMEM_SHARED((NP, F), jnp.float32),
            pltpu.VMEM((IB, CHUNK), jnp.int32),
            pltpu.VMEM((IB, CHUNK), jnp.int32),
            pltpu.VMEM((4, CHUNK, F), jnp.float32),
            pltpu.VMEM((ZROWS, F), jnp.float32),
            pltpu.SemaphoreType.DMA,
            pltpu.SemaphoreType.DMA,
            pltpu.SemaphoreType.DMA,
            pltpu.SemaphoreType.DMA,
            pltpu.SemaphoreType.DMA,
            pltpu.SemaphoreType.DMA,
            pltpu.SemaphoreType.DMA,
            pltpu.SemaphoreType.DMA,
        ],
    )(src2d, dst2d, y)


BLK_P = 1024
NM = NP // 4
BLK_E = 1568
BLK_M = 2000


def _prep_body(x_ref, pe_ref, deg_ref, et_ref, tw_ref, tb_ref, w1_ref,
               y1_ref, dx_ref):
    xb = x_ref[...].astype(jnp.int32)
    rows = lax.broadcasted_iota(jnp.int32, (28, BLK_P), 0)
    oh = (rows == xb).astype(jnp.float32)
    h0 = jnp.dot(et_ref[...], oh, preferred_element_type=jnp.float32)
    h0 += jnp.dot(tw_ref[...], pe_ref[...], preferred_element_type=jnp.float32)
    h0 += tb_ref[...]
    tT = jnp.dot(w1_ref[...], h0, preferred_element_type=jnp.float32)
    deg = jnp.sum(deg_ref[...], axis=0, keepdims=True) + 1.0
    dinv = lax.rsqrt(deg)
    y1T = tT * dinv
    dxT = jnp.broadcast_to(dinv, (F, BLK_P))

    def merge(vT):
        v3 = jnp.transpose(vT).reshape(BLK_P // 4, 4, F)
        return jnp.concatenate([v3[:, m, :] for m in range(4)], axis=1)

    y1_ref[...] = merge(y1T)
    dx_ref[...] = merge(dxT)


@functools.partial(jax.jit)
def _prep_kernel(xf, peT, deg_parts, EtT, TwT, tb_col, W1T):
    grid = NP // BLK_P
    return pl.pallas_call(
        _prep_body,
        grid=(grid,),
        in_specs=[
            pl.BlockSpec((1, BLK_P), lambda i: (0, i)),
            pl.BlockSpec((5, BLK_P), lambda i: (0, i)),
            pl.BlockSpec((NW, BLK_P), lambda i: (0, i)),
            pl.BlockSpec((64, 28), lambda i: (0, 0)),
            pl.BlockSpec((64, 5), lambda i: (0, 0)),
            pl.BlockSpec((64, 1), lambda i: (0, 0)),
            pl.BlockSpec((F, 64), lambda i: (0, 0)),
        ],
        out_specs=[
            pl.BlockSpec((BLK_P // 4, 128), lambda i: (i, 0)),
            pl.BlockSpec((BLK_P // 4, 128), lambda i: (i, 0)),
        ],
        out_shape=[
            jax.ShapeDtypeStruct((NM, 128), jnp.float32),
            jax.ShapeDtypeStruct((NM, 128), jnp.float32),
        ],
    )(xf, peT, deg_parts, EtT, TwT, tb_col, W1T)


def _ef_body(sp_ref, y1_ref, dx_ref, b1_ref, y2_ref):
    s = sp_ref[0] + sp_ref[1]
    dx = dx_ref[...]
    h1 = jnp.maximum(dx * (s + y1_ref[...]) + b1_ref[...], 0.0)
    y2_ref[...] = dx * h1


@functools.partial(jax.jit)
def _ef_kernel(s1m, y1m, dxm, b1_4):
    grid = NM // BLK_E
    return pl.pallas_call(
        _ef_body,
        grid=(grid,),
        in_specs=[
            pl.BlockSpec((NC, BLK_E, 128), lambda i: (0, i, 0)),
            pl.BlockSpec((BLK_E, 128), lambda i: (i, 0)),
            pl.BlockSpec((BLK_E, 128), lambda i: (i, 0)),
            pl.BlockSpec((1, 128), lambda i: (0, 0)),
        ],
        out_specs=pl.BlockSpec((BLK_E, 128), lambda i: (i, 0)),
        out_shape=jax.ShapeDtypeStruct((NM, 128), jnp.float32),
    )(s1m, y1m, dxm, b1_4)


def _mu_ls_body(sp_ref, y2_ref, dx_ref, wmu_ref, bmu_ref, wls_ref, bls_ref,
                mu_ref, ls_ref):
    agg = dx_ref[...] * (sp_ref[0] + sp_ref[1] + y2_ref[...])
    mu_ref[...] = jnp.dot(agg, wmu_ref[...],
                          preferred_element_type=jnp.float32) + bmu_ref[...]
    ls_ref[...] = jnp.dot(agg, wls_ref[...],
                          preferred_element_type=jnp.float32) + bls_ref[...]


@functools.partial(jax.jit)
def _mu_ls_kernel(s2m, y2m, dxm, W4mu, b4mu, W4ls, b4ls):
    grid = NM // BLK_E
    return pl.pallas_call(
        _mu_ls_body,
        grid=(grid,),
        in_specs=[
            pl.BlockSpec((NC, BLK_E, 128), lambda i: (0, i, 0)),
            pl.BlockSpec((BLK_E, 128), lambda i: (i, 0)),
            pl.BlockSpec((BLK_E, 128), lambda i: (i, 0)),
            pl.BlockSpec((128, 128), lambda i: (0, 0)),
            pl.BlockSpec((1, 128), lambda i: (0, 0)),
            pl.BlockSpec((128, 128), lambda i: (0, 0)),
            pl.BlockSpec((1, 128), lambda i: (0, 0)),
        ],
        out_specs=[
            pl.BlockSpec((BLK_E, 128), lambda i: (i, 0)),
            pl.BlockSpec((BLK_E, 128), lambda i: (i, 0)),
        ],
        out_shape=[
            jax.ShapeDtypeStruct((NM, 128), jnp.float32),
            jax.ShapeDtypeStruct((NM, 128), jnp.float32),
        ],
    )(s2m, y2m, dxm, W4mu, b4mu, W4ls, b4ls)


def kernel(x, edge_index, laplacian_eigenvector_pe, embed_table, trans_W,
           trans_b, W1, b1, W_mu, b_mu, W_ls, b_ls):
    xf = jnp.pad(x.reshape(-1).astype(jnp.float32), (0, NP - N))
    xf = xf.reshape(1, NP)
    peT = jnp.pad(laplacian_eigenvector_pe.astype(jnp.float32),
                  ((0, NP - N), (0, 0))).T
    ei = edge_index.astype(jnp.int32)
    pad_row = N + jnp.arange(EP - E, dtype=jnp.int32) % (NP - N)
    pad_e = jnp.broadcast_to(pad_row, (2, EP - E))
    e2 = jnp.concatenate([ei, pad_e], axis=1)
    src2d = e2[0].reshape(NW, NCHUNK, CHUNK)
    dst2d = e2[1].reshape(NW, NCHUNK, CHUNK)
    dst_flat = e2[1]

    EtT = embed_table.astype(jnp.float32).T
    TwT = trans_W.astype(jnp.float32).T
    tb_col = trans_b.astype(jnp.float32).reshape(64, 1)
    W1T = W1.astype(jnp.float32).T
    b1_4 = jnp.tile(b1.astype(jnp.float32), 4).reshape(1, 128)
    eye4 = jnp.eye(4, dtype=jnp.float32)
    W4mu = jnp.kron(eye4, W_mu.astype(jnp.float32))
    W4ls = jnp.kron(eye4, W_ls.astype(jnp.float32))
    b4mu = jnp.tile(b_mu.astype(jnp.float32), 4).reshape(1, 128)
    b4ls = jnp.tile(b_ls.astype(jnp.float32), 4).reshape(1, 128)

    deg_parts = _deg_kernel(dst_flat)
    y1m, dxm = _prep_kernel(xf, peT, deg_parts, EtT, TwT, tb_col, W1T)
    s1p = _scatter_kernel(src2d, dst2d, y1m.reshape(NP, F))
    y2m = _ef_kernel(s1p.reshape(NC, NM, 128), y1m, dxm, b1_4)
    s2p = _scatter_kernel(src2d, dst2d, y2m.reshape(NP, F))
    mu4, ls4 = _mu_ls_kernel(s2p.reshape(NC, NM, 128), y2m, dxm,
                             W4mu, b4mu, W4ls, b4ls)
    return (mu4.reshape(NP, F)[:N], ls4.reshape(NP, F)[:N])
